# baseline (speedup 1.0000x reference)
import jax
import jax.numpy as jnp
from jax import lax
from jax.experimental import pallas as pl
from jax.experimental.pallas import tpu as pltpu
from jax.experimental.shard_map import shard_map
from jax.sharding import Mesh, PartitionSpec as P

_L = 16
_NKH = 896
_HP = 128
_S = 384
_NCOL = _L * _NKH + _HP
_V_BLK = 256
_T_PER_CHUNK = 2


def _fused_body(slab_ref, wcombo_ref, mask_ref, wa_ref, b_ref, x_ref, out_ref):
    slab = slab_ref[...]
    mask = mask_ref[...]

    cw = _T_PER_CHUNK * _NKH
    pooled = None
    for c in range(_L // _T_PER_CHUNK):
        p = jnp.dot(slab, wcombo_ref[:, c * cw:(c + 1) * cw],
                    preferred_element_type=jnp.float32)
        for i in range(_T_PER_CHUNK):
            t = c * _T_PER_CHUNK + i
            cand = p[:, i * _NKH:(i + 1) * _NKH] + mask[t:t + 1, :]
            pooled = cand if pooled is None else jnp.maximum(pooled, cand)

    feat = jnp.tanh(pooled).astype(jnp.bfloat16)
    wproj = jnp.dot(slab, wcombo_ref[:, _L * _NKH:],
                    preferred_element_type=jnp.float32)

    y = jnp.tanh(jnp.dot(feat, wa_ref[...], preferred_element_type=jnp.float32)
                 + wproj + b_ref[...])

    out_ref[...] = lax.dot_general(x_ref[...], y, (((1,), (1,)), ((), ())),
                                   preferred_element_type=jnp.float32)


def _scores_local(slab, wcombo, mask, wa, bias, x32):
    Vl = slab.shape[0]
    B = x32.shape[0]
    n_blk = Vl // _V_BLK
    return pl.pallas_call(
        _fused_body,
        out_shape=jax.ShapeDtypeStruct((B, Vl), jnp.float32),
        grid=(n_blk,),
        in_specs=[
            pl.BlockSpec((_V_BLK, _S), lambda j: (j, 0)),
            pl.BlockSpec((_S, _NCOL), lambda j: (0, 0)),
            pl.BlockSpec((_L, _NKH), lambda j: (0, 0)),
            pl.BlockSpec((_NKH, _HP), lambda j: (0, 0)),
            pl.BlockSpec((1, _HP), lambda j: (0, 0)),
            pl.BlockSpec((B, _HP), lambda j: (0, 0)),
        ],
        out_specs=pl.BlockSpec((B, _V_BLK), lambda j: (0, j)),
        compiler_params=pltpu.CompilerParams(
            dimension_semantics=("arbitrary",),
            vmem_limit_bytes=50 * 1024 * 1024),
    )(slab, wcombo, mask, wa, bias, x32)


def kernel(slab, wcombo, mask, wa, bias, x):
    x32 = x.astype(jnp.float32)
    mesh = Mesh(jax.devices()[:2], ("v",))
    f = shard_map(
        _scores_local, mesh=mesh,
        in_specs=(P("v", None), P(None, None), P(None, None), P(None, None),
                  P(None, None), P(None, None)),
        out_specs=P(None, "v"), check_rep=False)
    out = f(slab, wcombo, mask, wa, bias, x32)
    return out[:, :40000]


# speedup vs baseline: 1.1512x; 1.1512x over previous
import jax
import jax.numpy as jnp
from jax import lax
from jax.experimental import pallas as pl
from jax.experimental.pallas import tpu as pltpu

_L = 16
_NKH = 896
_HP = 128
_S = 384
_NCOL = _L * _NKH + _HP
_V_OUT = 40000
_V_BLK = 1024
_T_PER_CHUNK = 2


def _fused_body(slab_ref, wcombo_ref, mask_ref, wa_ref, b_ref, x_ref, out_ref):
    slab = slab_ref[...]
    mask = mask_ref[...]

    cw = _T_PER_CHUNK * _NKH
    pooled = None
    for c in range(_L // _T_PER_CHUNK):
        p = jnp.dot(slab, wcombo_ref[:, c * cw:(c + 1) * cw],
                    preferred_element_type=jnp.float32)
        for i in range(_T_PER_CHUNK):
            t = c * _T_PER_CHUNK + i
            cand = p[:, i * _NKH:(i + 1) * _NKH] + mask[t:t + 1, :]
            pooled = cand if pooled is None else jnp.maximum(pooled, cand)

    feat = jnp.tanh(pooled).astype(jnp.bfloat16)
    wproj = jnp.dot(slab, wcombo_ref[:, _L * _NKH:],
                    preferred_element_type=jnp.float32)

    y = jnp.tanh(jnp.dot(feat, wa_ref[...], preferred_element_type=jnp.float32)
                 + wproj + b_ref[...])

    out_ref[...] = lax.dot_general(x_ref[...], y, (((1,), (1,)), ((), ())),
                                   preferred_element_type=jnp.float32)


def kernel(slab, wcombo, mask, wa, bias, x):
    B = x.shape[0]
    n_blk = -(-_V_OUT // _V_BLK)

    x32 = x.astype(jnp.float32)

    return pl.pallas_call(
        _fused_body,
        out_shape=jax.ShapeDtypeStruct((B, _V_OUT), jnp.float32),
        grid=(n_blk,),
        in_specs=[
            pl.BlockSpec((_V_BLK, _S), lambda j: (j, 0)),
            pl.BlockSpec((_S, _NCOL), lambda j: (0, 0)),
            pl.BlockSpec((_L, _NKH), lambda j: (0, 0)),
            pl.BlockSpec((_NKH, _HP), lambda j: (0, 0)),
            pl.BlockSpec((1, _HP), lambda j: (0, 0)),
            pl.BlockSpec((B, _HP), lambda j: (0, 0)),
        ],
        out_specs=pl.BlockSpec((B, _V_BLK), lambda j: (0, j)),
        compiler_params=pltpu.CompilerParams(
            dimension_semantics=("arbitrary",),
            vmem_limit_bytes=56 * 1024 * 1024),
    )(slab, wcombo, mask, wa, bias, x32)
